# Initial kernel scaffold; baseline (speedup 1.0000x reference)
import jax
import jax.numpy as jnp
from jax.experimental import pallas as pl


def kernel(feat, edge_index, bn_gamma, bn_beta, Wq, bq, Wk, Wv, We):
    raise NotImplementedError("write your pallas kernel here")


# speedup vs baseline: 2.3132x; 2.3132x over previous
import functools

import jax
import jax.numpy as jnp
from jax import lax
from jax.experimental import pallas as pl
from jax.experimental.pallas import tpu as pltpu
from jax.experimental.pallas import tpu_sc as plsc

_L = 16
_NC = 2
_NS = 16
_C = 80
_AW = 144


def _prep_body(feat_ref, g_ref, b_ref, wq_ref, bq_ref, wk_ref, wv_ref,
               q_ref, k_ref, v_ref):
    f = feat_ref[...]
    mu = jnp.mean(f, axis=0, keepdims=True)
    var = jnp.mean((f - mu) ** 2, axis=0, keepdims=True)
    x = (f - mu) * lax.rsqrt(var + 1e-5) * g_ref[...] + b_ref[...]
    dn = (((1,), (1,)), ((), ()))
    q_ref[...] = lax.dot_general(x, wq_ref[...], dn,
                                 precision=lax.Precision.HIGHEST) + bq_ref[...]
    k_ref[...] = lax.dot_general(x, wk_ref[...], dn,
                                 precision=lax.Precision.HIGHEST)
    v_ref[...] = lax.dot_general(x, wv_ref[...], dn,
                                 precision=lax.Precision.HIGHEST)


def _final_body(p_ref, o_ref):
    t = p_ref[0] + p_ref[1]
    s = t[:, 128:129]
    o_ref[...] = t[:, :128] / jnp.where(s > 0.0, s, 1.0)


def _make_sc_edge(N, E, D):
    per_tile = E // (_NC * _NS)
    chunks = per_tile // _C
    row_chunks = N // _C
    rr_iters = (row_chunks + _NS - 1) // _NS

    mesh = plsc.VectorSubcoreMesh(core_axis_name="c", subcore_axis_name="s",
                                  num_cores=_NC, num_subcores=_NS)

    @functools.partial(
        pl.kernel,
        out_type=jax.ShapeDtypeStruct((_NC, N, _AW), jnp.float32),
        mesh=mesh,
        scratch_types=[
            pltpu.VMEM((_C,), jnp.int32),
            pltpu.VMEM((_C,), jnp.int32),
            pltpu.VMEM((_C, D), jnp.float32),
            pltpu.VMEM((_C, D), jnp.float32),
            pltpu.VMEM((_C, _AW), jnp.float32),
            pltpu.VMEM((D,), jnp.float32),
            pltpu.VMEM_SHARED((N, _AW), jnp.float32),
            pltpu.SemaphoreType.DMA,
            pltpu.SemaphoreType.DMA,
        ],
        compiler_params=pltpu.CompilerParams(use_tc_tiling_on_sc=False,
                                             needs_layout_passes=False),
    )
    def sc_edge(q_hbm, k_hbm, v_hbm, edge_hbm, we_hbm, out_hbm,
                src_v, dst_v, qb, kb, ob, web, acc, sem_a, sem_b):
        cid = lax.axis_index("c")
        sid = lax.axis_index("s")
        lane = lax.iota(jnp.int32, _L)
        zero16 = jnp.zeros((_L,), jnp.float32)

        def zrow(r, _):
            for cc in range(_AW // _L):
                ob[r, pl.ds(cc * _L, _L)] = zero16
            return 0
        lax.fori_loop(0, _C, zrow, 0)
        for jj in range(rr_iters):
            c = jj * _NS + sid
            @pl.when(c < row_chunks)
            def _():
                pltpu.sync_copy(ob, acc.at[pl.ds(c * _C, _C)])

        pltpu.sync_copy(we_hbm.at[0], web)
        plsc.subcore_barrier()

        ebase = (cid * _NS + sid) * per_tile

        def chunk(i, _):
            base = ebase + i * _C
            pltpu.sync_copy(edge_hbm.at[pl.ds(base, _C)], src_v)
            pltpu.sync_copy(edge_hbm.at[pl.ds(E + base, _C)], dst_v)
            cp_q = pltpu.async_copy(q_hbm.at[src_v], qb, sem_a)
            cp_k = pltpu.async_copy(k_hbm.at[dst_v], kb, sem_b)
            cp_q.wait()
            cp_k.wait()

            def group_a(p, _):
                rows = p * _L + lane

                def hstep(hh, accv):
                    for u in range(8):
                        h = hh * 8 + u
                        hv = jnp.broadcast_to(h, (_L,))
                        qvals = plsc.load_gather(qb, [rows, hv])
                        kvals = plsc.load_gather(kb, [rows, hv])
                        wh = plsc.load_gather(web, [hv])
                        sg = 1.0 / (1.0 + jnp.exp(-(qvals + kvals)))
                        accv = accv + wh * sg
                    return accv
                accv = lax.fori_loop(0, D // 8, hstep, zero16)
                w = jnp.exp(accv)
                plsc.store_scatter(
                    ob, [rows, jnp.broadcast_to(jnp.int32(D), (_L,))], w)
                return 0
            lax.fori_loop(0, _C // _L, group_a, 0)

            pltpu.async_copy(v_hbm.at[src_v], qb, sem_a).wait()

            def group_b(p, _):
                rows = p * _L + lane
                w = plsc.load_gather(
                    ob, [rows, jnp.broadcast_to(jnp.int32(D), (_L,))])

                def cstep(cc, _):
                    for u in range(8):
                        c = cc * 8 + u
                        cv = jnp.broadcast_to(c, (_L,))
                        vvals = plsc.load_gather(qb, [rows, cv])
                        plsc.store_scatter(ob, [rows, cv], vvals * w)
                    return 0
                lax.fori_loop(0, D // 8, cstep, 0)
                return 0
            lax.fori_loop(0, _C // _L, group_b, 0)

            pltpu.sync_copy(ob, acc.at[dst_v], add=True)
            return 0
        lax.fori_loop(0, chunks, chunk, 0)

        plsc.subcore_barrier()
        for jj in range(rr_iters):
            c = jj * _NS + sid
            @pl.when(c < row_chunks)
            def _():
                sl = pl.ds(c * _C, _C)
                pltpu.sync_copy(acc.at[sl], out_hbm.at[cid, sl])

    return sc_edge


def kernel(feat, edge_index, bn_gamma, bn_beta, Wq, bq, Wk, Wv, We):
    N, D = feat.shape
    E = edge_index.shape[1]
    q, k, v = pl.pallas_call(
        _prep_body,
        out_shape=(
            jax.ShapeDtypeStruct((N, D), jnp.float32),
            jax.ShapeDtypeStruct((N, D), jnp.float32),
            jax.ShapeDtypeStruct((N, D), jnp.float32),
        ),
    )(feat, bn_gamma.reshape(1, -1), bn_beta.reshape(1, -1),
      Wq, bq.reshape(1, -1), Wk, Wv)

    partials = _make_sc_edge(N, E, D)(q, k, v, edge_index.reshape(-1), We)

    rst = pl.pallas_call(
        _final_body,
        out_shape=jax.ShapeDtypeStruct((N, D), jnp.float32),
    )(partials)
    return rst
